# baseline (speedup 1.0000x reference)
import functools

import jax
import jax.numpy as jnp
from jax import lax
from jax.experimental import pallas as pl
from jax.experimental.pallas import tpu as pltpu
from jax.experimental.pallas import tpu_sc as plsc

_INFO = plsc.get_sparse_core_info()
_NC = _INFO.num_cores
_NS = _INFO.num_subcores
_NW = _NC * _NS
_LANES = _INFO.num_lanes

_NBUF = 2
_PADW = 128
_H0 = 104


@functools.lru_cache(maxsize=None)
def _build(B, L, D, V):
    assert B % _NW == 0 and D % _LANES == 0
    seq_per_w = B // _NW
    assert seq_per_w % _NBUF == 0
    c0 = min(128, L)
    c1 = L - c0
    nvec = D // _LANES
    row_blk = 8
    h0 = min(_H0, L)
    h1 = L - h0
    assert h0 % row_blk == 0 and h1 % row_blk == 0
    halves = ((0, h0), (h0, h1)) if h1 else ((0, h0),)

    mesh = plsc.VectorSubcoreMesh(core_axis_name="c", subcore_axis_name="s")

    @functools.partial(
        pl.kernel,
        out_type=jax.ShapeDtypeStruct((B, L, D), jnp.float32),
        mesh=mesh,
        scratch_types=[
            [pltpu.VMEM((L,), jnp.int32)] * _NBUF,
            pltpu.VMEM((L, D), jnp.float32),
            [pltpu.VMEM((L, _PADW), jnp.float32)] * _NBUF,
            [pltpu.VMEM((h, D), jnp.float32) for _, h in halves],
            [pltpu.SemaphoreType.DMA] * _NBUF,
            [pltpu.SemaphoreType.DMA] * _NBUF,
            [pltpu.SemaphoreType.DMA] * len(halves),
        ],
    )
    def emb_kernel(x_hbm, tok_hbm, pos_hbm, out_hbm, idx_bufs, pos_v,
                   tok_bufs, out_bufs, isems, gsems, osems):
        cid = lax.axis_index("c")
        sid = lax.axis_index("s")
        wid = sid * _NC + cid
        seq0 = wid * seq_per_w

        pltpu.sync_copy(pos_hbm, pos_v)

        def start_idx(s, b):
            pltpu.async_copy(x_hbm.at[pl.ds((seq0 + s) * L, L)],
                             idx_bufs[b], isems[b])

        def wait_idx(b):
            pltpu.make_async_copy(x_hbm.at[pl.ds(0, L)], idx_bufs[b],
                                  isems[b]).wait()

        def start_gather(b):
            tb = tok_bufs[b]
            iv = idx_bufs[b]
            pltpu.async_copy(tok_hbm.at[iv.at[pl.ds(0, c0)]],
                             tb.at[pl.ds(0, c0)], gsems[b])
            if c1:
                pltpu.async_copy(tok_hbm.at[iv.at[pl.ds(c0, c1)]],
                                 tb.at[pl.ds(c0, c1)], gsems[b])

        def wait_gather(b):
            tb = tok_bufs[b]
            iv = idx_bufs[b]
            pltpu.make_async_copy(tok_hbm.at[iv.at[pl.ds(0, c0)]],
                                  tb.at[pl.ds(0, c0)], gsems[b]).wait()
            if c1:
                pltpu.make_async_copy(tok_hbm.at[iv.at[pl.ds(c0, c1)]],
                                      tb.at[pl.ds(c0, c1)], gsems[b]).wait()

        def start_out(s, h):
            base, rows = halves[h]
            pltpu.async_copy(out_bufs[h],
                             out_hbm.at[seq0 + s, pl.ds(base, rows)],
                             osems[h])

        def wait_out(h):
            base, rows = halves[h]
            pltpu.make_async_copy(out_bufs[h],
                                  out_hbm.at[seq0, pl.ds(base, rows)],
                                  osems[h]).wait()

        def add_pos(b, h):
            tb = tok_bufs[b]
            ob = out_bufs[h]
            base, rows = halves[h]

            @pl.loop(0, rows, step=row_blk)
            def _(r):
                for rr in range(row_blk):
                    for j in range(nvec):
                        sl = pl.ds(j * _LANES, _LANES)
                        ob[r + rr, sl] = (tb[base + r + rr, sl]
                                          + pos_v[base + r + rr, sl])

        start_idx(0, 0)
        wait_idx(0)
        start_gather(0)
        start_idx(1, 1)

        @pl.loop(0, seq_per_w, step=_NBUF)
        def _(g):
            for b in range(_NBUF):
                s = g + b
                nb = (b + 1) % _NBUF

                @pl.when(s + 1 < seq_per_w)
                def _():
                    wait_idx(nb)
                    start_gather(nb)

                wait_gather(b)

                @pl.when(s + 2 < seq_per_w)
                def _():
                    start_idx(s + 2, b)

                for h in range(len(halves)):
                    @pl.when(s >= 1)
                    def _():
                        wait_out(h)

                    add_pos(b, h)
                    start_out(s, h)

        for h in range(len(halves)):
            wait_out(h)

    return emb_kernel


@functools.lru_cache(maxsize=None)
def _build_pad(V, D):
    rows = 1000
    assert V % rows == 0

    def pad_body(t_ref, o_ref):
        o_ref[:, :D] = t_ref[...]

    return pl.pallas_call(
        pad_body,
        grid=(V // rows,),
        in_specs=[pl.BlockSpec((rows, D), lambda i: (i, 0))],
        out_specs=pl.BlockSpec((rows, _PADW), lambda i: (i, 0)),
        out_shape=jax.ShapeDtypeStruct((V, _PADW), jnp.float32),
    )


def kernel(x, token_table, pos_table):
    B, L = x.shape
    V, D = token_table.shape
    fn = _build(B, L, D, V)
    tt = _build_pad(V, D)(token_table)
    return fn(x.astype(jnp.int32).reshape(B * L), tt, pos_table)


# speedup vs baseline: 1.2507x; 1.2507x over previous
import functools

import jax
import jax.numpy as jnp
from jax import lax
from jax.experimental import pallas as pl
from jax.experimental.pallas import tpu as pltpu
from jax.experimental.pallas import tpu_sc as plsc

_INFO = plsc.get_sparse_core_info()
_NC = _INFO.num_cores
_NS = _INFO.num_subcores
_NW = _NC * _NS
_LANES = _INFO.num_lanes

_PADW = 128
_H0 = 104


@functools.lru_cache(maxsize=None)
def _build(B, L, D, V):
    assert B % _NW == 0 and D % _LANES == 0
    seq_per_w = B // _NW
    assert seq_per_w % 2 == 0
    nvec = D // _LANES
    row_blk = 8
    h0 = min(_H0, L)
    h1 = L - h0
    assert h0 % row_blk == 0 and (h1 == 0 or h1 % row_blk == 0)
    halves = ((0, h0), (h0, h1)) if h1 else ((0, h0),)
    nh = len(halves)
    hmax = max(h for _, h in halves)

    mesh = plsc.VectorSubcoreMesh(core_axis_name="c", subcore_axis_name="s")

    @functools.partial(
        pl.kernel,
        out_type=jax.ShapeDtypeStruct((B, L, D), jnp.float32),
        mesh=mesh,
        scratch_types=[
            [pltpu.VMEM((L,), jnp.int32)] * 2,
            pltpu.VMEM((L, D), jnp.float32),
            [pltpu.VMEM((hmax, _PADW), jnp.float32)] * (2 * nh),
            [pltpu.VMEM((hmax, D), jnp.float32)] * nh,
            [pltpu.SemaphoreType.DMA] * 2,
            [pltpu.SemaphoreType.DMA] * (2 * nh),
            [pltpu.SemaphoreType.DMA] * nh,
        ],
    )
    def emb_kernel(x_hbm, tok_hbm, pos_hbm, out_hbm, idx_bufs, pos_v,
                   tok_bufs, out_bufs, isems, gsems, osems):
        cid = lax.axis_index("c")
        sid = lax.axis_index("s")
        wid = sid * _NC + cid
        seq0 = wid * seq_per_w

        pltpu.sync_copy(pos_hbm, pos_v)

        def start_idx(s, ib):
            pltpu.async_copy(x_hbm.at[pl.ds((seq0 + s) * L, L)],
                             idx_bufs[ib], isems[ib])

        def wait_idx(ib):
            pltpu.make_async_copy(x_hbm.at[pl.ds(0, L)], idx_bufs[ib],
                                  isems[ib]).wait()

        def start_gather(ib, h, tb):
            base, rows = halves[h]
            pltpu.async_copy(
                tok_hbm.at[idx_bufs[ib].at[pl.ds(base, rows)]],
                tok_bufs[tb].at[pl.ds(0, rows)], gsems[tb])

        def wait_gather(h, tb):
            base, rows = halves[h]
            pltpu.make_async_copy(
                tok_hbm.at[idx_bufs[0].at[pl.ds(0, rows)]],
                tok_bufs[tb].at[pl.ds(0, rows)], gsems[tb]).wait()

        def start_out(s, h):
            base, rows = halves[h]
            pltpu.async_copy(out_bufs[h].at[pl.ds(0, rows)],
                             out_hbm.at[seq0 + s, pl.ds(base, rows)],
                             osems[h])

        def wait_out(h):
            base, rows = halves[h]
            pltpu.make_async_copy(out_bufs[h].at[pl.ds(0, rows)],
                                  out_hbm.at[seq0, pl.ds(base, rows)],
                                  osems[h]).wait()

        def add_pos(h, tb):
            t = tok_bufs[tb]
            ob = out_bufs[h]
            base, rows = halves[h]

            @pl.loop(0, rows, step=row_blk)
            def _(r):
                for rr in range(row_blk):
                    for j in range(nvec):
                        sl = pl.ds(j * _LANES, _LANES)
                        ob[r + rr, sl] = (t[r + rr, sl]
                                          + pos_v[base + r + rr, sl])

        start_idx(0, 0)
        wait_idx(0)
        for h in range(nh):
            start_gather(0, h, h)
        start_idx(1, 1)

        @pl.loop(0, seq_per_w, step=2)
        def _(g):
            for b in range(2):
                s = g + b
                ib = b
                nib = 1 - b
                tbs = [(2 * s0 + h) % (2 * nh)
                       for s0, h in ((b, 0), (b, 1), (b + 1, 0), (b + 1, 1))]

                @pl.when(s + 1 < seq_per_w)
                def _():
                    wait_idx(nib)
                    for h in range(nh):
                        start_gather(nib, h, tbs[2 + h])

                for h in range(nh):
                    wait_gather(h, tbs[h])

                @pl.when(s + 2 < seq_per_w)
                def _():
                    start_idx(s + 2, ib)

                for h in range(nh):
                    @pl.when(s >= 1)
                    def _():
                        wait_out(h)

                    add_pos(h, tbs[h])
                    start_out(s, h)

        for h in range(nh):
            wait_out(h)

    return emb_kernel


def kernel(x, token_table, pos_table):
    B, L = x.shape
    V, D = token_table.shape
    fn = _build(B, L, D, V)
    tt = jnp.pad(token_table, ((0, 0), (0, _PADW - D)))
    return fn(x.astype(jnp.int32).reshape(B * L), tt, pos_table)
